# baseline (speedup 1.0000x reference)
import functools

import jax
import jax.numpy as jnp
from jax import lax
from jax.experimental import pallas as pl
from jax.experimental.pallas import tpu as pltpu
from jax.experimental.pallas import tpu_sc as plsc

N_ANNO = 800000
K = 8
D = 4
N_ANNOT = 1000
N_ITEMS = 50000

NC = 2
NS = 16
NW = NC * NS
L = 16

C = 1280
NB = C // 128
VPC = C // L
NCHUNKS = N_ANNO // C
BIG_W = NCHUNKS - (NCHUNKS // NW) * NW

IPAD = 51200
ROWS_PT = IPAD // NS
TBLK = IPAD // 128
TBLK_PT = ROWS_PT // 128

MIN_LL = -13.815510557964274
LN2 = 0.6931471805599453
EBIAS = 127 * LN2


def _fast_log(x):
    bits = lax.bitcast_convert_type(x, jnp.int32)
    ef = lax.shift_right_arithmetic(bits, 23).astype(jnp.float32)
    m = lax.bitcast_convert_type(
        (bits & 0x007FFFFF) | 0x3F800000, jnp.float32)
    z = (m - 1.0) / (m + 1.0)
    z2 = z * z
    a = ((2.0 / 7.0) * z2 + (2.0 / 5.0)) * z2 + (2.0 / 3.0)
    a = a * z2 + 2.0
    return (z * a - EBIAS) + ef * LN2


def _body(mu_hbm, re_hbm, conf_hbm, anno_hbm, ann_hbm, items_hbm,
          w_hbm, tot_hbm,
          re_v, mu_v, e_v, f_v, c_v,
          annb, anob, cfb, idxb, wb, sb, zb, zt, acc):
    cid_core = lax.axis_index("c")
    sid = lax.axis_index("s")
    wid = sid * NC + cid_core

    iota = lax.iota(jnp.int32, L)

    pltpu.sync_copy(re_hbm, re_v)
    pltpu.sync_copy(mu_hbm, mu_v.at[pl.ds(0, K * D)])

    mu0 = mu_v[pl.ds(0, L)]
    mu1 = mu_v[pl.ds(L, L)]
    E0 = jnp.exp(mu0)
    E1 = jnp.exp(mu1)
    e_v[pl.ds(0, L)] = E0
    e_v[pl.ds(L, L)] = E1

    OFF = 8
    grp = iota & -D
    for h, Eh in ((0, E0), (1, E1)):
        b = grp + h * L
        q = [plsc.load_gather(e_v, [b + d]) for d in range(D)]
        M = jnp.maximum(jnp.maximum(q[0], q[1]), jnp.maximum(q[2], q[3]))
        Ch = Eh - M
        f_v[pl.ds(OFF + h * L, L)] = jnp.exp(Ch)
        c_v[pl.ds(OFF + h * L, L)] = Ch

    F = [[plsc.load_gather(f_v, [jnp.full((L,), OFF + k * D + d, jnp.int32)])
          for d in range(D)] for k in range(K)]

    @plsc.parallel_loop(0, ROWS_PT * K // L, 1, unroll=4)
    def _zero(i):
        fl = iota + i * L
        plsc.store_scatter(zb, [lax.shift_right_arithmetic(fl, 3), fl & 7],
                           jnp.zeros((L,), jnp.float32))
    r0 = sid * ROWS_PT
    pltpu.sync_copy(zb, acc.at[pl.ds(r0, ROWS_PT)])
    plsc.subcore_barrier()

    def vec_body(i):
        off = i * L
        ann = annb[pl.ds(off, L)]
        an = anob[pl.ds(off, L)]
        cf = cfb[pl.ds(off, L)]
        a4 = ann * D
        r = [plsc.load_gather(re_v, [a4 + d]) for d in range(D)]
        g = [jnp.exp(rd) for rd in r]
        ra = plsc.load_gather(re_v, [a4 + an])
        nrow = iota + off
        for k in range(K):
            ck = plsc.load_gather(c_v, [an + (8 + k * D)])
            s = F[k][0] * g[0] + F[k][1] * g[1] + F[k][2] * g[2] + F[k][3] * g[3]
            ll = ck + ra - _fast_log(s)
            w = jnp.maximum(ll, MIN_LL) * cf
            wb[lax.shift_right_arithmetic(i, 3), k, pl.ds((i & 7) * L, L)] = w
            plsc.store_scatter(sb, [nrow, jnp.full((L,), k, jnp.int32)], w)

    def chunk_body(t, _):
        cid = wid + t * NW
        base = cid * C
        pltpu.sync_copy(ann_hbm.at[pl.ds(base, C)], annb)
        pltpu.sync_copy(anno_hbm.at[pl.ds(base, C)], anob)
        pltpu.sync_copy(conf_hbm.at[pl.ds(base, C)], cfb)
        for j in range(NB):
            pltpu.sync_copy(items_hbm.at[pl.ds(base + j * 128, 128)],
                            idxb.at[j])
        plsc.parallel_loop(0, VPC, 1, unroll=2)(vec_body)
        pltpu.sync_copy(wb, w_hbm.at[pl.ds(cid * NB, NB)])
        for j in range(NB):
            pltpu.sync_copy(sb.at[pl.ds(j * 128, 128)],
                            acc.at[idxb.at[j]], add=True)
        return 0

    nt = jnp.where(wid < BIG_W, NCHUNKS // NW + 1, NCHUNKS // NW)
    lax.fori_loop(0, nt, chunk_body, 0)

    plsc.subcore_barrier()
    pltpu.sync_copy(acc.at[pl.ds(r0, ROWS_PT)], zb)

    @plsc.parallel_loop(0, ROWS_PT // L, 1, unroll=2)
    def _tr(v):
        rvec = iota + v * L
        for k in range(K):
            col = plsc.load_gather(zb, [rvec, jnp.full((L,), k, jnp.int32)])
            zt[lax.shift_right_arithmetic(v, 3), k, pl.ds((v & 7) * L, L)] = col
    pltpu.sync_copy(zt, tot_hbm.at[cid_core, pl.ds(sid * TBLK_PT, TBLK_PT)])


@functools.partial(
    pl.kernel,
    out_type=(
        jax.ShapeDtypeStruct((N_ANNO // 128, K, 128), jnp.float32),
        jax.ShapeDtypeStruct((NC, TBLK, K, 128), jnp.float32),
    ),
    mesh=plsc.VectorSubcoreMesh(core_axis_name="c", subcore_axis_name="s"),
    compiler_params=pltpu.CompilerParams(needs_layout_passes=False,
                                         use_tc_tiling_on_sc=False),
    scratch_types=[
        pltpu.VMEM((N_ANNOT * D,), jnp.float32),
        pltpu.VMEM((128,), jnp.float32),
        pltpu.VMEM((128,), jnp.float32),
        pltpu.VMEM((128,), jnp.float32),
        pltpu.VMEM((128,), jnp.float32),
        pltpu.VMEM((C,), jnp.int32),
        pltpu.VMEM((C,), jnp.int32),
        pltpu.VMEM((C,), jnp.float32),
        pltpu.VMEM((NB, 128), jnp.int32),
        pltpu.VMEM((NB, K, 128), jnp.float32),
        pltpu.VMEM((C, K), jnp.float32),
        pltpu.VMEM((ROWS_PT, K), jnp.float32),
        pltpu.VMEM((TBLK_PT, K, 128), jnp.float32),
        pltpu.VMEM_SHARED((IPAD, K), jnp.float32),
    ],
)
def _sc_likelihood(mu_hbm, re_hbm, conf_hbm, anno_hbm, ann_hbm, items_hbm,
                   w_hbm, tot_hbm, *scratch):
    _body(mu_hbm, re_hbm, conf_hbm, anno_hbm, ann_hbm, items_hbm,
          w_hbm, tot_hbm, *scratch)


def kernel(mu, random_effects, confidence, anno, annotators, items):
    w_blk, tot_blk = _sc_likelihood(
        mu.reshape(K * D),
        random_effects.reshape(N_ANNOT * D),
        confidence,
        anno,
        annotators,
        items,
    )
    weighted = w_blk.transpose(1, 0, 2).reshape(K, N_ANNO)
    tot = tot_blk[0] + tot_blk[1]
    total_ll = tot.transpose(1, 0, 2).reshape(K, IPAD)[:, :N_ITEMS]
    return weighted, total_ll


# speedup vs baseline: 1.7950x; 1.7950x over previous
import functools

import jax
import jax.numpy as jnp
from jax import lax
from jax.experimental import pallas as pl
from jax.experimental.pallas import tpu as pltpu
from jax.experimental.pallas import tpu_sc as plsc

N_ANNO = 800000
K = 8
D = 4
N_ANNOT = 1000
N_ITEMS = 50000

NC = 2
NS = 16
NW = NC * NS
L = 16

C = 1280
NB = C // 128
VPC = C // L
NCHUNKS = N_ANNO // C
BIG_W = NCHUNKS - (NCHUNKS // NW) * NW

IPAD = 51200
ROWS_PT = IPAD // NS
TBLK = IPAD // 128
TBLK_PT = ROWS_PT // 128

MIN_LL = -13.815510557964274
LN2 = 0.6931471805599453
EBIAS = 127 * LN2


def _fast_log(x):
    bits = lax.bitcast_convert_type(x, jnp.int32)
    ef = lax.shift_right_arithmetic(bits, 23).astype(jnp.float32)
    m = lax.bitcast_convert_type(
        (bits & 0x007FFFFF) | 0x3F800000, jnp.float32)
    z = (m - 1.0) / (m + 1.0)
    z2 = z * z
    a = ((2.0 / 7.0) * z2 + (2.0 / 5.0)) * z2 + (2.0 / 3.0)
    a = a * z2 + 2.0
    return (z * a - EBIAS) + ef * LN2


def _body(mu_hbm, re_hbm, conf_hbm, anno_hbm, ann_hbm, items_hbm,
          w_hbm, tot_hbm,
          re_v, mu_v, e_v, f_v, c_v,
          annb, anob, cfb, idxb, wb, sb, zb, zt, acc, insem, wsem, ssem):
    cid_core = lax.axis_index("c")
    sid = lax.axis_index("s")
    wid = sid * NC + cid_core

    iota = lax.iota(jnp.int32, L)

    pltpu.sync_copy(re_hbm, re_v)
    pltpu.sync_copy(mu_hbm, mu_v.at[pl.ds(0, K * D)])

    mu0 = mu_v[pl.ds(0, L)]
    mu1 = mu_v[pl.ds(L, L)]
    E0 = jnp.exp(mu0)
    E1 = jnp.exp(mu1)
    e_v[pl.ds(0, L)] = E0
    e_v[pl.ds(L, L)] = E1

    OFF = 8
    grp = iota & -D
    for h, Eh in ((0, E0), (1, E1)):
        b = grp + h * L
        q = [plsc.load_gather(e_v, [b + d]) for d in range(D)]
        M = jnp.maximum(jnp.maximum(q[0], q[1]), jnp.maximum(q[2], q[3]))
        Ch = Eh - M
        f_v[pl.ds(OFF + h * L, L)] = jnp.exp(Ch)
        c_v[pl.ds(OFF + h * L, L)] = Ch

    F = [[plsc.load_gather(f_v, [jnp.full((L,), OFF + k * D + d, jnp.int32)])
          for d in range(D)] for k in range(K)]

    @plsc.parallel_loop(0, ROWS_PT * K // L, 1, unroll=4)
    def _zero(i):
        fl = iota + i * L
        plsc.store_scatter(zb, [lax.shift_right_arithmetic(fl, 3), fl & 7],
                           jnp.zeros((L,), jnp.float32))
    r0 = sid * ROWS_PT
    pltpu.sync_copy(zb, acc.at[pl.ds(r0, ROWS_PT)])
    plsc.subcore_barrier()

    def vec_body(i):
        off = i * L
        ann = annb[pl.ds(off, L)]
        an = anob[pl.ds(off, L)]
        cf = cfb[pl.ds(off, L)]
        a4 = ann * D
        r = [plsc.load_gather(re_v, [a4 + d]) for d in range(D)]
        g = [jnp.exp(rd) for rd in r]
        ra = plsc.load_gather(re_v, [a4 + an])
        nrow = iota + off
        for k in range(K):
            ck = plsc.load_gather(c_v, [an + (8 + k * D)])
            s = F[k][0] * g[0] + F[k][1] * g[1] + F[k][2] * g[2] + F[k][3] * g[3]
            ll = ck + ra - _fast_log(s)
            w = jnp.maximum(ll, MIN_LL) * cf
            wb[lax.shift_right_arithmetic(i, 3), k, pl.ds((i & 7) * L, L)] = w
            plsc.store_scatter(sb, [nrow, jnp.full((L,), k, jnp.int32)], w)

    def chunk_body(t, _):
        cid = wid + t * NW
        base = cid * C
        ins = [
            pltpu.async_copy(ann_hbm.at[pl.ds(base, C)], annb, insem),
            pltpu.async_copy(anno_hbm.at[pl.ds(base, C)], anob, insem),
            pltpu.async_copy(conf_hbm.at[pl.ds(base, C)], cfb, insem),
            pltpu.async_copy(items_hbm.at[pl.ds(cid * NB, NB)], idxb, insem),
        ]
        for d in ins:
            d.wait()
        plsc.parallel_loop(0, VPC, 1, unroll=2)(vec_body)
        outs = [pltpu.async_copy(wb, w_hbm.at[pl.ds(cid * NB, NB)], wsem)]
        outs += [
            pltpu.async_copy(sb.at[pl.ds(j * 128, 128)],
                             acc.at[idxb.at[j]], ssem, add=True)
            for j in range(NB)
        ]
        for d in outs:
            d.wait()
        return 0

    nt = jnp.where(wid < BIG_W, NCHUNKS // NW + 1, NCHUNKS // NW)
    lax.fori_loop(0, nt, chunk_body, 0)

    plsc.subcore_barrier()
    pltpu.sync_copy(acc.at[pl.ds(r0, ROWS_PT)], zb)

    @plsc.parallel_loop(0, ROWS_PT // L, 1, unroll=2)
    def _tr(v):
        rvec = iota + v * L
        for k in range(K):
            col = plsc.load_gather(zb, [rvec, jnp.full((L,), k, jnp.int32)])
            zt[lax.shift_right_arithmetic(v, 3), k, pl.ds((v & 7) * L, L)] = col
    pltpu.sync_copy(zt, tot_hbm.at[cid_core, pl.ds(sid * TBLK_PT, TBLK_PT)])


@functools.partial(
    pl.kernel,
    out_type=(
        jax.ShapeDtypeStruct((N_ANNO // 128, K, 128), jnp.float32),
        jax.ShapeDtypeStruct((NC, TBLK, K, 128), jnp.float32),
    ),
    mesh=plsc.VectorSubcoreMesh(core_axis_name="c", subcore_axis_name="s"),
    compiler_params=pltpu.CompilerParams(needs_layout_passes=False,
                                         use_tc_tiling_on_sc=False),
    scratch_types=[
        pltpu.VMEM((N_ANNOT * D,), jnp.float32),
        pltpu.VMEM((128,), jnp.float32),
        pltpu.VMEM((128,), jnp.float32),
        pltpu.VMEM((128,), jnp.float32),
        pltpu.VMEM((128,), jnp.float32),
        pltpu.VMEM((C,), jnp.int32),
        pltpu.VMEM((C,), jnp.int32),
        pltpu.VMEM((C,), jnp.float32),
        pltpu.VMEM((NB, 128), jnp.int32),
        pltpu.VMEM((NB, K, 128), jnp.float32),
        pltpu.VMEM((C, K), jnp.float32),
        pltpu.VMEM((ROWS_PT, K), jnp.float32),
        pltpu.VMEM((TBLK_PT, K, 128), jnp.float32),
        pltpu.VMEM_SHARED((IPAD, K), jnp.float32),
        pltpu.SemaphoreType.DMA,
        pltpu.SemaphoreType.DMA,
        pltpu.SemaphoreType.DMA,
    ],
)
def _sc_likelihood(mu_hbm, re_hbm, conf_hbm, anno_hbm, ann_hbm, items_hbm,
                   w_hbm, tot_hbm, *scratch):
    _body(mu_hbm, re_hbm, conf_hbm, anno_hbm, ann_hbm, items_hbm,
          w_hbm, tot_hbm, *scratch)


def kernel(mu, random_effects, confidence, anno, annotators, items):
    w_blk, tot_blk = _sc_likelihood(
        mu.reshape(K * D),
        random_effects.reshape(N_ANNOT * D),
        confidence,
        anno,
        annotators,
        items.reshape(N_ANNO // 128, 128),
    )
    weighted = w_blk.transpose(1, 0, 2).reshape(K, N_ANNO)
    tot = tot_blk[0] + tot_blk[1]
    total_ll = tot.transpose(1, 0, 2).reshape(K, IPAD)[:, :N_ITEMS]
    return weighted, total_ll


# speedup vs baseline: 2.0193x; 1.1250x over previous
import functools

import jax
import jax.numpy as jnp
from jax import lax
from jax.experimental import pallas as pl
from jax.experimental.pallas import tpu as pltpu
from jax.experimental.pallas import tpu_sc as plsc

N_ANNO = 800000
K = 8
D = 4
N_ANNOT = 1000
N_ITEMS = 50000

NC = 2
NS = 16
NW = NC * NS
L = 16

C = 1280
NB = C // 128
VPC = C // L
NCHUNKS = N_ANNO // C
NT = (NCHUNKS + NW - 1) // NW

IPAD = 51200
ROWS_PT = IPAD // NS
TBLK = IPAD // 128
TBLK_PT = ROWS_PT // 128

MIN_LL = -13.815510557964274
LN2 = 0.6931471805599453
EBIAS = 127 * LN2


def _fast_log(x):
    bits = lax.bitcast_convert_type(x, jnp.int32)
    ef = lax.shift_right_arithmetic(bits, 23).astype(jnp.float32)
    m = lax.bitcast_convert_type(
        (bits & 0x007FFFFF) | 0x3F800000, jnp.float32)
    z = (m - 1.0) / (m + 1.0)
    z2 = z * z
    a = ((2.0 / 7.0) * z2 + (2.0 / 5.0)) * z2 + (2.0 / 3.0)
    a = a * z2 + 2.0
    return (z * a - EBIAS) + ef * LN2


def _body(mu_hbm, re_hbm, conf_hbm, anno_hbm, ann_hbm, items_hbm,
          w_hbm, tot_hbm,
          re_v, mu_v, e_v, f_v, c_v,
          annb, anob, cfb, idxb, wb, sb, zb, zt, acc,
          insem0, insem1, wsem0, wsem1, ssem0, ssem1):
    cid_core = lax.axis_index("c")
    sid = lax.axis_index("s")
    wid = sid * NC + cid_core

    iota = lax.iota(jnp.int32, L)

    pltpu.sync_copy(re_hbm, re_v)
    pltpu.sync_copy(mu_hbm, mu_v.at[pl.ds(0, K * D)])

    mu0 = mu_v[pl.ds(0, L)]
    mu1 = mu_v[pl.ds(L, L)]
    E0 = jnp.exp(mu0)
    E1 = jnp.exp(mu1)
    e_v[pl.ds(0, L)] = E0
    e_v[pl.ds(L, L)] = E1

    OFF = 8
    grp = iota & -D
    for h, Eh in ((0, E0), (1, E1)):
        b = grp + h * L
        q = [plsc.load_gather(e_v, [b + d]) for d in range(D)]
        M = jnp.maximum(jnp.maximum(q[0], q[1]), jnp.maximum(q[2], q[3]))
        Ch = Eh - M
        f_v[pl.ds(OFF + h * L, L)] = jnp.exp(Ch)
        c_v[pl.ds(OFF + h * L, L)] = Ch

    F = [[plsc.load_gather(f_v, [jnp.full((L,), OFF + k * D + d, jnp.int32)])
          for d in range(D)] for k in range(K)]

    HR = ROWS_PT // 2

    @plsc.parallel_loop(0, HR * K // L, 1, unroll=4)
    def _zero(i):
        fl = iota + i * L
        plsc.store_scatter(zb, [lax.shift_right_arithmetic(fl, 3), fl & 7],
                           jnp.zeros((L,), jnp.float32))
    r0 = sid * ROWS_PT
    pltpu.sync_copy(zb, acc.at[pl.ds(r0, HR)])
    pltpu.sync_copy(zb, acc.at[pl.ds(r0 + HR, HR)])
    plsc.subcore_barrier()

    def make_vec_body(wbb, sbb, annbb, anobb, cfbb):
        def vec_body(i):
            off = i * L
            ann = annbb[pl.ds(off, L)]
            an = anobb[pl.ds(off, L)]
            cf = cfbb[pl.ds(off, L)]
            a4 = ann * D
            r = [plsc.load_gather(re_v, [a4 + d]) for d in range(D)]
            g = [jnp.exp(rd) for rd in r]
            ra = plsc.load_gather(re_v, [a4 + an])
            nrow = iota + off
            for k in range(K):
                ck = plsc.load_gather(c_v, [an + (8 + k * D)])
                s = (F[k][0] * g[0] + F[k][1] * g[1]
                     + F[k][2] * g[2] + F[k][3] * g[3])
                ll = ck + ra - _fast_log(s)
                w = jnp.maximum(ll, MIN_LL) * cf
                wbb[lax.shift_right_arithmetic(i, 3), k,
                    pl.ds((i & 7) * L, L)] = w
                plsc.store_scatter(sbb, [nrow, jnp.full((L,), k, jnp.int32)],
                                   w)
        return vec_body

    insems = (insem0, insem1)
    wsems = (wsem0, wsem1)
    ssems = (ssem0, ssem1)

    def in_triples(t, b):
        cid = wid + t * NW
        base = cid * C
        isem = insems[b % 2]
        return [
            (ann_hbm.at[pl.ds(base, C)], annb.at[b % 2], isem),
            (anno_hbm.at[pl.ds(base, C)], anob.at[b % 2], isem),
            (conf_hbm.at[pl.ds(base, C)], cfb.at[b % 2], isem),
            (items_hbm.at[pl.ds(cid * NB, NB)], idxb.at[b % 4], isem),
        ]

    def w_triple(t, b):
        cid = wid + t * NW
        return (wb.at[b % 2], w_hbm.at[pl.ds(cid * NB, NB)], wsems[b % 2])

    def s_triples(b):
        return [
            (sb.at[b % 2, pl.ds(j * 128, 128)], acc.at[idxb.at[b % 4, j]],
             ssems[b % 2])
            for j in range(NB)
        ]

    def fire_ins(t, b):
        for s_, d_, m_ in in_triples(t, b):
            pltpu.async_copy(s_, d_, m_)

    fire_ins(0, 0)
    fire_ins(1, 1)

    def outer(s, _):
        for b4 in range(4):
            t = s * 4 + b4
            cid = wid + t * NW
            real = cid < NCHUNKS
            prev_real = (t >= 2) & (cid - 2 * NW < NCHUNKS)

            @pl.when(prev_real)
            def _wait_prev():
                sw, dw, mw = w_triple(t - 2, b4 + 2)
                pltpu.make_async_copy(sw, dw, mw).wait()
                for s_, d_, m_ in s_triples(b4 + 2):
                    pltpu.make_async_copy(s_, d_, m_).wait()

            @pl.when(real)
            def _do_chunk():
                for s_, d_, m_ in in_triples(t, b4):
                    pltpu.make_async_copy(s_, d_, m_).wait()
                plsc.parallel_loop(0, VPC, 1, unroll=2)(
                    make_vec_body(wb.at[b4 % 2], sb.at[b4 % 2],
                                  annb.at[b4 % 2], anob.at[b4 % 2],
                                  cfb.at[b4 % 2]))
                sw, dw, mw = w_triple(t, b4)
                pltpu.async_copy(sw, dw, mw)
                for s_, d_, m_ in s_triples(b4):
                    pltpu.async_copy(s_, d_, m_, add=True)

            nxt_real = (t + 2 < NT) & (cid + 2 * NW < NCHUNKS)

            @pl.when(nxt_real)
            def _prefetch():
                fire_ins(t + 2, b4 + 2)
        return 0

    lax.fori_loop(0, NT // 4, outer, 0)

    for b4 in (2, 3):
        t = NT - 4 + b4
        cid = wid + t * NW

        @pl.when(cid < NCHUNKS)
        def _drain():
            sw, dw, mw = w_triple(t, b4)
            pltpu.make_async_copy(sw, dw, mw).wait()
            for s_, d_, m_ in s_triples(b4):
                pltpu.make_async_copy(s_, d_, m_).wait()

    plsc.subcore_barrier()
    for p in range(2):
        pltpu.sync_copy(acc.at[pl.ds(r0 + p * HR, HR)], zb)

        @plsc.parallel_loop(0, HR // L, 1, unroll=2)
        def _tr(v, p=p):
            rvec = iota + v * L
            vg = v + p * (HR // L)
            for k in range(K):
                col = plsc.load_gather(
                    zb, [rvec, jnp.full((L,), k, jnp.int32)])
                zt[lax.shift_right_arithmetic(vg, 3), k,
                   pl.ds((vg & 7) * L, L)] = col
    pltpu.sync_copy(zt, tot_hbm.at[cid_core, pl.ds(sid * TBLK_PT, TBLK_PT)])


@functools.partial(
    pl.kernel,
    out_type=(
        jax.ShapeDtypeStruct((N_ANNO // 128, K, 128), jnp.float32),
        jax.ShapeDtypeStruct((NC, TBLK, K, 128), jnp.float32),
    ),
    mesh=plsc.VectorSubcoreMesh(core_axis_name="c", subcore_axis_name="s"),
    compiler_params=pltpu.CompilerParams(needs_layout_passes=False,
                                         use_tc_tiling_on_sc=False),
    scratch_types=[
        pltpu.VMEM((N_ANNOT * D,), jnp.float32),
        pltpu.VMEM((128,), jnp.float32),
        pltpu.VMEM((128,), jnp.float32),
        pltpu.VMEM((128,), jnp.float32),
        pltpu.VMEM((128,), jnp.float32),
        pltpu.VMEM((2, C), jnp.int32),
        pltpu.VMEM((2, C), jnp.int32),
        pltpu.VMEM((2, C), jnp.float32),
        pltpu.VMEM((4, NB, 128), jnp.int32),
        pltpu.VMEM((2, NB, K, 128), jnp.float32),
        pltpu.VMEM((2, C, K), jnp.float32),
        pltpu.VMEM((ROWS_PT // 2, K), jnp.float32),
        pltpu.VMEM((TBLK_PT, K, 128), jnp.float32),
        pltpu.VMEM_SHARED((IPAD, K), jnp.float32),
        pltpu.SemaphoreType.DMA,
        pltpu.SemaphoreType.DMA,
        pltpu.SemaphoreType.DMA,
        pltpu.SemaphoreType.DMA,
        pltpu.SemaphoreType.DMA,
        pltpu.SemaphoreType.DMA,
    ],
)
def _sc_likelihood(mu_hbm, re_hbm, conf_hbm, anno_hbm, ann_hbm, items_hbm,
                   w_hbm, tot_hbm, *scratch):
    _body(mu_hbm, re_hbm, conf_hbm, anno_hbm, ann_hbm, items_hbm,
          w_hbm, tot_hbm, *scratch)


def kernel(mu, random_effects, confidence, anno, annotators, items):
    w_blk, tot_blk = _sc_likelihood(
        mu.reshape(K * D),
        random_effects.reshape(N_ANNOT * D),
        confidence,
        anno,
        annotators,
        items.reshape(N_ANNO // 128, 128),
    )
    weighted = w_blk.transpose(1, 0, 2).reshape(K, N_ANNO)
    tot = tot_blk[0] + tot_blk[1]
    total_ll = tot.transpose(1, 0, 2).reshape(K, IPAD)[:, :N_ITEMS]
    return weighted, total_ll


# speedup vs baseline: 2.0895x; 1.0348x over previous
import functools

import jax
import jax.numpy as jnp
from jax import lax
from jax.experimental import pallas as pl
from jax.experimental.pallas import tpu as pltpu
from jax.experimental.pallas import tpu_sc as plsc

N_ANNO = 800000
K = 8
D = 4
N_ANNOT = 1000
N_ITEMS = 50000

NC = 2
NS = 16
NW = NC * NS
L = 16

C = 1280
NB = C // 128
VPC = C // L
NCHUNKS = N_ANNO // C
NT = (NCHUNKS + NW - 1) // NW

IPAD = 51200
ROWS_PT = IPAD // NS
TBLK = IPAD // 128
TBLK_PT = ROWS_PT // 128

MIN_LL = -13.815510557964274
LN2 = 0.6931471805599453
EBIAS = 127 * LN2


def _fast_log(x):
    bits = lax.bitcast_convert_type(x, jnp.int32)
    ef = lax.shift_right_arithmetic(bits, 23).astype(jnp.float32)
    m = lax.bitcast_convert_type(
        (bits & 0x007FFFFF) | 0x3F800000, jnp.float32)
    z = (m - 1.0) / (m + 1.0)
    z2 = z * z
    a = ((2.0 / 7.0) * z2 + (2.0 / 5.0)) * z2 + (2.0 / 3.0)
    a = a * z2 + 2.0
    return z * a + ef * LN2


def _body(mu_hbm, re_hbm, conf_hbm, anno_hbm, ann_hbm, items_hbm,
          w_hbm, tot_hbm,
          re_v, mu_v, e_v, f_v, c_v,
          annb, anob, cfb, idxb, wb, sb, zb, zt, acc,
          insem0, insem1, wsem0, wsem1, ssem0, ssem1):
    cid_core = lax.axis_index("c")
    sid = lax.axis_index("s")
    wid = sid * NC + cid_core

    iota = lax.iota(jnp.int32, L)

    pltpu.sync_copy(re_hbm, re_v)
    pltpu.sync_copy(mu_hbm, mu_v.at[pl.ds(0, K * D)])

    mu0 = mu_v[pl.ds(0, L)]
    mu1 = mu_v[pl.ds(L, L)]
    E0 = jnp.exp(mu0)
    E1 = jnp.exp(mu1)
    e_v[pl.ds(0, L)] = E0
    e_v[pl.ds(L, L)] = E1

    OFF = 8
    grp = iota & -D
    for h, Eh in ((0, E0), (1, E1)):
        b = grp + h * L
        q = [plsc.load_gather(e_v, [b + d]) for d in range(D)]
        M = jnp.maximum(jnp.maximum(q[0], q[1]), jnp.maximum(q[2], q[3]))
        Ch = Eh - M
        f_v[pl.ds(OFF + h * L, L)] = jnp.exp(Ch)
        c_v[pl.ds(OFF + h * L, L)] = Ch + EBIAS

    F = [[plsc.load_gather(f_v, [jnp.full((L,), OFF + k * D + d, jnp.int32)])
          for d in range(D)] for k in range(K)]

    HR = ROWS_PT // 2

    @plsc.parallel_loop(0, HR * K // L, 1, unroll=4)
    def _zero(i):
        fl = iota + i * L
        plsc.store_scatter(zb, [lax.shift_right_arithmetic(fl, 3), fl & 7],
                           jnp.zeros((L,), jnp.float32))
    r0 = sid * ROWS_PT
    pltpu.sync_copy(zb, acc.at[pl.ds(r0, HR)])
    pltpu.sync_copy(zb, acc.at[pl.ds(r0 + HR, HR)])
    plsc.subcore_barrier()

    def make_vec_body(wbb, sbb, annbb, anobb, cfbb):
        def vec_body(i):
            off = i * L
            ann = annbb[pl.ds(off, L)]
            an = anobb[pl.ds(off, L)]
            cf = cfbb[pl.ds(off, L)]
            a4 = ann * D
            r = [plsc.load_gather(re_v, [a4 + d]) for d in range(D)]
            g = [jnp.exp(rd) for rd in r]
            ra = plsc.load_gather(re_v, [a4 + an])
            nrow = iota + off
            for k in range(K):
                ck = plsc.load_gather(c_v, [an + (8 + k * D)])
                s = (F[k][0] * g[0] + F[k][1] * g[1]
                     + F[k][2] * g[2] + F[k][3] * g[3])
                ll = ck + ra - _fast_log(s)
                w = jnp.maximum(ll, MIN_LL) * cf
                wbb[lax.shift_right_arithmetic(i, 3), k,
                    pl.ds((i & 7) * L, L)] = w
                plsc.store_scatter(sbb, [nrow, jnp.full((L,), k, jnp.int32)],
                                   w)
        return vec_body

    insems = (insem0, insem1)
    wsems = (wsem0, wsem1)
    ssems = (ssem0, ssem1)

    def in_triples(t, b):
        cid = wid + t * NW
        base = cid * C
        isem = insems[b % 2]
        return [
            (ann_hbm.at[pl.ds(base, C)], annb.at[b % 2], isem),
            (anno_hbm.at[pl.ds(base, C)], anob.at[b % 2], isem),
            (conf_hbm.at[pl.ds(base, C)], cfb.at[b % 2], isem),
            (items_hbm.at[pl.ds(cid * NB, NB)], idxb.at[b % 4], isem),
        ]

    def w_triple(t, b):
        cid = wid + t * NW
        return (wb.at[b % 2], w_hbm.at[pl.ds(cid * NB, NB)], wsems[b % 2])

    def s_triples(b):
        return [
            (sb.at[b % 2, pl.ds(j * 128, 128)], acc.at[idxb.at[b % 4, j]],
             ssems[b % 2])
            for j in range(NB)
        ]

    def fire_ins(t, b):
        for s_, d_, m_ in in_triples(t, b):
            pltpu.async_copy(s_, d_, m_)

    fire_ins(0, 0)
    fire_ins(1, 1)

    def outer(s, _):
        for b4 in range(4):
            t = s * 4 + b4
            cid = wid + t * NW
            real = cid < NCHUNKS
            prev_real = (t >= 2) & (cid - 2 * NW < NCHUNKS)

            @pl.when(prev_real)
            def _wait_prev():
                sw, dw, mw = w_triple(t - 2, b4 + 2)
                pltpu.make_async_copy(sw, dw, mw).wait()
                for s_, d_, m_ in s_triples(b4 + 2):
                    pltpu.make_async_copy(s_, d_, m_).wait()

            @pl.when(real)
            def _do_chunk():
                for s_, d_, m_ in in_triples(t, b4):
                    pltpu.make_async_copy(s_, d_, m_).wait()
                plsc.parallel_loop(0, VPC, 1, unroll=2)(
                    make_vec_body(wb.at[b4 % 2], sb.at[b4 % 2],
                                  annb.at[b4 % 2], anob.at[b4 % 2],
                                  cfb.at[b4 % 2]))
                sw, dw, mw = w_triple(t, b4)
                pltpu.async_copy(sw, dw, mw)
                for s_, d_, m_ in s_triples(b4):
                    pltpu.async_copy(s_, d_, m_, add=True)

            nxt_real = (t + 2 < NT) & (cid + 2 * NW < NCHUNKS)

            @pl.when(nxt_real)
            def _prefetch():
                fire_ins(t + 2, b4 + 2)
        return 0

    lax.fori_loop(0, NT // 4, outer, 0)

    for b4 in (2, 3):
        t = NT - 4 + b4
        cid = wid + t * NW

        @pl.when(cid < NCHUNKS)
        def _drain():
            sw, dw, mw = w_triple(t, b4)
            pltpu.make_async_copy(sw, dw, mw).wait()
            for s_, d_, m_ in s_triples(b4):
                pltpu.make_async_copy(s_, d_, m_).wait()

    plsc.subcore_barrier()
    for p in range(2):
        pltpu.sync_copy(acc.at[pl.ds(r0 + p * HR, HR)], zb)

        @plsc.parallel_loop(0, HR // L, 1, unroll=2)
        def _tr(v, p=p):
            rvec = iota + v * L
            vg = v + p * (HR // L)
            for k in range(K):
                col = plsc.load_gather(
                    zb, [rvec, jnp.full((L,), k, jnp.int32)])
                zt[lax.shift_right_arithmetic(vg, 3), k,
                   pl.ds((vg & 7) * L, L)] = col
    pltpu.sync_copy(zt, tot_hbm.at[cid_core, pl.ds(sid * TBLK_PT, TBLK_PT)])


@functools.partial(
    pl.kernel,
    out_type=(
        jax.ShapeDtypeStruct((N_ANNO // 128, K, 128), jnp.float32),
        jax.ShapeDtypeStruct((NC, TBLK, K, 128), jnp.float32),
    ),
    mesh=plsc.VectorSubcoreMesh(core_axis_name="c", subcore_axis_name="s"),
    compiler_params=pltpu.CompilerParams(needs_layout_passes=False,
                                         use_tc_tiling_on_sc=False),
    scratch_types=[
        pltpu.VMEM((N_ANNOT * D,), jnp.float32),
        pltpu.VMEM((128,), jnp.float32),
        pltpu.VMEM((128,), jnp.float32),
        pltpu.VMEM((128,), jnp.float32),
        pltpu.VMEM((128,), jnp.float32),
        pltpu.VMEM((2, C), jnp.int32),
        pltpu.VMEM((2, C), jnp.int32),
        pltpu.VMEM((2, C), jnp.float32),
        pltpu.VMEM((4, NB, 128), jnp.int32),
        pltpu.VMEM((2, NB, K, 128), jnp.float32),
        pltpu.VMEM((2, C, K), jnp.float32),
        pltpu.VMEM((ROWS_PT // 2, K), jnp.float32),
        pltpu.VMEM((TBLK_PT, K, 128), jnp.float32),
        pltpu.VMEM_SHARED((IPAD, K), jnp.float32),
        pltpu.SemaphoreType.DMA,
        pltpu.SemaphoreType.DMA,
        pltpu.SemaphoreType.DMA,
        pltpu.SemaphoreType.DMA,
        pltpu.SemaphoreType.DMA,
        pltpu.SemaphoreType.DMA,
    ],
)
def _sc_likelihood(mu_hbm, re_hbm, conf_hbm, anno_hbm, ann_hbm, items_hbm,
                   w_hbm, tot_hbm, *scratch):
    _body(mu_hbm, re_hbm, conf_hbm, anno_hbm, ann_hbm, items_hbm,
          w_hbm, tot_hbm, *scratch)


def kernel(mu, random_effects, confidence, anno, annotators, items):
    w_blk, tot_blk = _sc_likelihood(
        mu.reshape(K * D),
        random_effects.reshape(N_ANNOT * D),
        confidence,
        anno,
        annotators,
        items.reshape(N_ANNO // 128, 128),
    )
    weighted = w_blk.transpose(1, 0, 2).reshape(K, N_ANNO)
    tot = tot_blk[0] + tot_blk[1]
    total_ll = tot.transpose(1, 0, 2).reshape(K, IPAD)[:, :N_ITEMS]
    return weighted, total_ll
